# Initial kernel scaffold; baseline (speedup 1.0000x reference)
import jax
import jax.numpy as jnp
from jax.experimental import pallas as pl


def kernel(positions, species, senders, receivers, l0_W1, l0_R0, l0_R1, l0_R2, l0_W2, l0_Wskip, l1_W1, l1_R0, l1_R1, l1_R2, l1_W2, l1_Wskip, W_ro, atom_energies):
    raise NotImplementedError("write your pallas kernel here")


# speedup vs baseline: 2.0440x; 2.0440x over previous
import functools

import jax
import jax.numpy as jnp
import numpy as np
from jax import lax
from jax.experimental import pallas as pl
from jax.experimental.pallas import tpu as pltpu
from jax.experimental.pallas import tpu_sc as plsc

N_SPECIES = 8
HIDDEN = 64
RB = 8
RM = 64
CUTOFF = 5.0
AVG_N = 16.0
SCALE = 1.0
SHIFT = 0.0
N = 50000
E = 800000

N_PAD = 50176
EP = 819200
TRASH = N

NC = 2
NS = 16
G = 1024
GC = G // 128

F0 = 16
F1 = 32

BN = 1792
BE = 2048


def _silu(x):
    return x * (1.0 / (1.0 + jnp.exp(-x)))


def _sc_mesh():
    return plsc.VectorSubcoreMesh(core_axis_name="c", subcore_axis_name="s")


def _geom_call(posx, posy, posz, senders, receivers):
    ET = EP // (NC * NS)
    NG = ET // G

    def body(px_hbm, py_hbm, pz_hbm, snd_hbm, rcv_hbm, sq_hbm,
             sidx, ridx, xs, ys, zs, xr, yr, zr, sqv, sem):
        c = lax.axis_index("c")
        s = lax.axis_index("s")
        wid = s * NC + c
        base0 = wid * ET

        def group(g, carry):
            base = base0 + g * G
            pltpu.sync_copy(snd_hbm.at[pl.ds(base, G)], sidx)
            pltpu.sync_copy(rcv_hbm.at[pl.ds(base, G)], ridx)
            cps = [
                pltpu.async_copy(px_hbm.at[sidx], xs, sem),
                pltpu.async_copy(py_hbm.at[sidx], ys, sem),
                pltpu.async_copy(pz_hbm.at[sidx], zs, sem),
                pltpu.async_copy(px_hbm.at[ridx], xr, sem),
                pltpu.async_copy(py_hbm.at[ridx], yr, sem),
                pltpu.async_copy(pz_hbm.at[ridx], zr, sem),
            ]
            for cp in cps:
                cp.wait()

            def sub(i, carry2):
                sl = pl.ds(i * 16, 16)
                dx = xs[sl] - xr[sl]
                dy = ys[sl] - yr[sl]
                dz = zs[sl] - zr[sl]
                sqv[sl] = dx * dx + dy * dy + dz * dz
                return carry2

            lax.fori_loop(0, G // 16, sub, 0)
            pltpu.sync_copy(sqv, sq_hbm.at[pl.ds(base, G)])
            return carry

        lax.fori_loop(0, NG, group, 0)

    k = pl.kernel(
        body,
        out_type=jax.ShapeDtypeStruct((EP,), jnp.float32),
        mesh=_sc_mesh(),
        compiler_params=pltpu.CompilerParams(use_tc_tiling_on_sc=False),
        scratch_types=[
            pltpu.VMEM((G,), jnp.int32),
            pltpu.VMEM((G,), jnp.int32),
            pltpu.VMEM((G,), jnp.float32),
            pltpu.VMEM((G,), jnp.float32),
            pltpu.VMEM((G,), jnp.float32),
            pltpu.VMEM((G,), jnp.float32),
            pltpu.VMEM((G,), jnp.float32),
            pltpu.VMEM((G,), jnp.float32),
            pltpu.VMEM((G,), jnp.float32),
            pltpu.SemaphoreType.DMA,
        ],
    )
    return k(posx, posy, posz, senders, receivers)


def _msgpass_call(node_tab, senders, recv2d, rad, zeros_nf, width, edge_split,
                  G=G):
    GC = G // 128
    ET = (EP // 2 if edge_split else EP) // NS
    SUPER = 1024
    NSUB = SUPER // G
    NG = ET // SUPER
    ROWS_T = N_PAD // NS
    WCH = max(w for w in (784, 448, 224, 112, 56)
              if w <= G and ROWS_T % w == 0)
    NWC = ROWS_T // WCH

    def body(node_hbm, snd_hbm, rcv_hbm, rad_hbm, zero_hbm, out_hbm,
             sidx, ridx2, rows, radv, shared, sem):
        c = lax.axis_index("c")
        s = lax.axis_index("s")

        for k2 in range(NWC):
            roff = s * ROWS_T + k2 * WCH
            pltpu.sync_copy(zero_hbm.at[pl.ds(roff, WCH)],
                            shared.at[pl.ds(roff, WCH)])
        plsc.subcore_barrier()

        if edge_split:
            ebase = c * (EP // 2) + s * ET
            rad_base0 = ebase
        else:
            ebase = s * ET
            rad_base0 = c * EP + s * ET

        def group(g, carry):
            base = ebase + g * SUPER
            rbase = rad_base0 + g * SUPER
            pltpu.sync_copy(snd_hbm.at[pl.ds(base, SUPER)], sidx)
            pltpu.sync_copy(
                rcv_hbm.at[pl.ds(pl.multiple_of(base // 128, 8), SUPER // 128)],
                ridx2)
            if not edge_split:
                off = c * N_PAD

                def addoff(i, carry2):
                    sidx[pl.ds(i * 16, 16)] = sidx[pl.ds(i * 16, 16)] + off
                    return carry2

                lax.fori_loop(0, SUPER // 16, addoff, 0)
            for sub in range(NSUB):
                gcp = pltpu.async_copy(
                    node_hbm.at[sidx.at[pl.ds(sub * G, G)]], rows, sem)
                pltpu.sync_copy(rad_hbm.at[pl.ds(rbase + sub * G, G)], radv)
                gcp.wait()

                def mul(i, carry2):
                    for j in range(width // 16):
                        sl = pl.ds(j * 16, 16)
                        rows[i, sl] = rows[i, sl] * radv[i, sl]
                    return carry2

                lax.fori_loop(0, G, mul, 0)
                for j in range(GC):
                    pltpu.sync_copy(rows.at[pl.ds(j * 128, 128)],
                                    shared.at[ridx2.at[sub * GC + j]],
                                    add=True)
            return carry

        lax.fori_loop(0, NG, group, 0)
        plsc.subcore_barrier()
        for k2 in range(NWC):
            roff = s * ROWS_T + k2 * WCH
            pltpu.sync_copy(shared.at[pl.ds(roff, WCH)],
                            rows.at[pl.ds(0, WCH)])
            pltpu.sync_copy(rows.at[pl.ds(0, WCH)],
                            out_hbm.at[pl.ds(c * N_PAD + roff, WCH)])

    k = pl.kernel(
        body,
        out_type=jax.ShapeDtypeStruct((2 * N_PAD, width), jnp.float32),
        mesh=_sc_mesh(),
        compiler_params=pltpu.CompilerParams(use_tc_tiling_on_sc=False),
        scratch_types=[
            pltpu.VMEM((SUPER,), jnp.int32),
            pltpu.VMEM((SUPER // 128, 128), jnp.int32),
            pltpu.VMEM((G, width), jnp.float32),
            pltpu.VMEM((G, width), jnp.float32),
            pltpu.VMEM_SHARED((N_PAD, width), jnp.float32),
            pltpu.SemaphoreType.DMA,
        ],
    )
    return k(node_tab, senders, recv2d, rad, zeros_nf)


def _radial_body(sq_ref, r00, r01, r02, r10, r11, r12, rad0_ref, rad1_ref):
    sq = sq_ref[...]
    iszero = sq == 0.0
    rn = jnp.where(iszero, 0.0, jnp.sqrt(jnp.where(iszero, 1.0, sq)))
    wmat = (lax.broadcasted_iota(jnp.int32, (BE, RB), 1) + 1
            ).astype(jnp.float32) * np.pi
    x_safe = jnp.where(rn == 0.0, 1.0, rn)
    bes = (2.0 / CUTOFF) * jnp.where(
        rn == 0.0, wmat / CUTOFF, jnp.sin(wmat * x_safe / CUTOFF) / x_safe)
    xh = rn * (1.0 / CUTOFF)
    x2 = xh * xh
    cut = (1.0 - 6.0 * x2 + 8.0 * x2 * xh - 3.0 * x2 * x2)
    cut = cut * jnp.where(xh < 1.0, 1.0, 0.0)
    rb = bes * cut

    def mlp(a, b, cc):
        h = _silu(jnp.dot(rb, a[...], preferred_element_type=jnp.float32))
        h = _silu(jnp.dot(h, b[...], preferred_element_type=jnp.float32))
        return jnp.dot(h, cc[...], preferred_element_type=jnp.float32)

    r0 = mlp(r00, r01, r02)
    rad0_ref[...] = jnp.pad(r0, ((0, 0), (0, F0 - RB)))
    r1 = mlp(r10, r11, r12)
    rad1_ref[0] = r1[:, :F1]
    rad1_ref[1] = r1[:, F1:]


def _radial_call(sq2d, R00, R01, R02, R10, R11, R12):
    grid = (EP // BE,)
    return pl.pallas_call(
        _radial_body,
        grid=grid,
        in_specs=[
            pl.BlockSpec((BE, 1), lambda i: (i, 0)),
            pl.BlockSpec((RB, RM), lambda i: (0, 0)),
            pl.BlockSpec((RM, RM), lambda i: (0, 0)),
            pl.BlockSpec((RM, RB), lambda i: (0, 0)),
            pl.BlockSpec((RB, RM), lambda i: (0, 0)),
            pl.BlockSpec((RM, RM), lambda i: (0, 0)),
            pl.BlockSpec((RM, HIDDEN), lambda i: (0, 0)),
        ],
        out_specs=[
            pl.BlockSpec((BE, F0), lambda i: (i, 0)),
            pl.BlockSpec((2, BE, F1), lambda i: (0, i, 0)),
        ],
        out_shape=[
            jax.ShapeDtypeStruct((EP, F0), jnp.float32),
            jax.ShapeDtypeStruct((2, EP, F1), jnp.float32),
        ],
    )(sq2d, R00, R01, R02, R10, R11, R12)


def _node0_body(spec_ref, w1_ref, out_ref):
    spec = spec_ref[...]
    oh = (spec == lax.broadcasted_iota(jnp.int32, (BN, N_SPECIES), 1))
    oh = oh.astype(jnp.float32)
    n0 = jnp.dot(oh, w1_ref[...], preferred_element_type=jnp.float32)
    out_ref[...] = jnp.pad(n0, ((0, 0), (0, F0 - N_SPECIES)))


def _node0_call(spec2d, W1_0):
    return pl.pallas_call(
        _node0_body,
        grid=(N_PAD // BN,),
        in_specs=[
            pl.BlockSpec((BN, 1), lambda i: (i, 0)),
            pl.BlockSpec((N_SPECIES, N_SPECIES), lambda i: (0, 0)),
        ],
        out_specs=pl.BlockSpec((BN, F0), lambda i: (i, 0)),
        out_shape=jax.ShapeDtypeStruct((N_PAD, F0), jnp.float32),
    )(spec2d, W1_0)


def _comb0_body(part_ref, spec_ref, skiptab_ref, w2_ref, w1n_ref,
                feat1_ref, node1_ref):
    p = part_ref[0] + part_ref[1]
    agg = p[:, :N_SPECIES] * (1.0 / np.sqrt(AVG_N))
    spec = spec_ref[...]
    oh = (spec == lax.broadcasted_iota(jnp.int32, (BN, N_SPECIES), 1))
    oh = oh.astype(jnp.float32)
    skip = jnp.dot(oh, skiptab_ref[...], preferred_element_type=jnp.float32)
    f1 = _silu(jnp.dot(agg, w2_ref[...],
                       preferred_element_type=jnp.float32) + skip)
    feat1_ref[...] = f1
    n1 = jnp.dot(f1, w1n_ref[...], preferred_element_type=jnp.float32)
    node1_ref[0] = n1[:, :F1]
    node1_ref[1] = n1[:, F1:]


def _comb0_call(part0, spec2d, skip0_tab, W2_0, W1_1):
    return pl.pallas_call(
        _comb0_body,
        grid=(N_PAD // BN,),
        in_specs=[
            pl.BlockSpec((2, BN, F0), lambda i: (0, i, 0)),
            pl.BlockSpec((BN, 1), lambda i: (i, 0)),
            pl.BlockSpec((N_SPECIES, HIDDEN), lambda i: (0, 0)),
            pl.BlockSpec((N_SPECIES, HIDDEN), lambda i: (0, 0)),
            pl.BlockSpec((HIDDEN, HIDDEN), lambda i: (0, 0)),
        ],
        out_specs=[
            pl.BlockSpec((BN, HIDDEN), lambda i: (i, 0)),
            pl.BlockSpec((2, BN, F1), lambda i: (0, i, 0)),
        ],
        out_shape=[
            jax.ShapeDtypeStruct((N_PAD, HIDDEN), jnp.float32),
            jax.ShapeDtypeStruct((2, N_PAD, F1), jnp.float32),
        ],
    )(part0, spec2d, skip0_tab, W2_0, W1_1)


def _comb1_body(agg_ref, feat1_ref, spec_ref, wall_ref, w2_ref, wro_ref,
                ae_ref, out_ref):
    agg = jnp.concatenate([agg_ref[0], agg_ref[1]], axis=1)
    agg = agg * (1.0 / np.sqrt(AVG_N))
    f1 = feat1_ref[...]
    hs = jnp.dot(f1, wall_ref[...], preferred_element_type=jnp.float32)
    spec = spec_ref[...]
    skip = jnp.zeros((BN, HIDDEN), jnp.float32)
    for s in range(N_SPECIES):
        skip = skip + jnp.where(spec == s,
                                hs[:, s * HIDDEN:(s + 1) * HIDDEN], 0.0)
    f2 = _silu(jnp.dot(agg, w2_ref[...],
                       preferred_element_type=jnp.float32) + skip)
    e = jnp.dot(f2, wro_ref[...], preferred_element_type=jnp.float32)
    e = e * SCALE + SHIFT
    oh = (spec == lax.broadcasted_iota(jnp.int32, (BN, N_SPECIES), 1))
    e = e + jnp.dot(oh.astype(jnp.float32), ae_ref[...],
                    preferred_element_type=jnp.float32)
    out_ref[...] = e


def _comb1_call(agg1, feat1, spec2d, W_all, W2_1, W_ro, ae2d):
    return pl.pallas_call(
        _comb1_body,
        grid=(N_PAD // BN,),
        in_specs=[
            pl.BlockSpec((2, BN, F1), lambda i: (0, i, 0)),
            pl.BlockSpec((BN, HIDDEN), lambda i: (i, 0)),
            pl.BlockSpec((BN, 1), lambda i: (i, 0)),
            pl.BlockSpec((HIDDEN, N_SPECIES * HIDDEN), lambda i: (0, 0)),
            pl.BlockSpec((HIDDEN, HIDDEN), lambda i: (0, 0)),
            pl.BlockSpec((HIDDEN, 1), lambda i: (0, 0)),
            pl.BlockSpec((N_SPECIES, 1), lambda i: (0, 0)),
        ],
        out_specs=pl.BlockSpec((BN, 1), lambda i: (i, 0)),
        out_shape=jax.ShapeDtypeStruct((N_PAD, 1), jnp.float32),
    )(agg1, feat1, spec2d, W_all, W2_1, W_ro, ae2d)


def kernel(positions, species, senders, receivers,
           l0_W1, l0_R0, l0_R1, l0_R2, l0_W2, l0_Wskip,
           l1_W1, l1_R0, l1_R1, l1_R2, l1_W2, l1_Wskip,
           W_ro, atom_energies):
    f32 = jnp.float32
    posx = jnp.zeros((N_PAD,), f32).at[:N].set(positions[:, 0].astype(f32))
    posy = jnp.zeros((N_PAD,), f32).at[:N].set(positions[:, 1].astype(f32))
    posz = jnp.zeros((N_PAD,), f32).at[:N].set(positions[:, 2].astype(f32))
    spec_p = jnp.zeros((N_PAD,), jnp.int32).at[:N].set(
        species.astype(jnp.int32))
    snd_p = jnp.full((EP,), TRASH, jnp.int32).at[:E].set(
        senders.astype(jnp.int32))
    rcv_p = jnp.full((EP,), TRASH, jnp.int32).at[:E].set(
        receivers.astype(jnp.int32))
    rcv2d = rcv_p.reshape(EP // 128, 128)
    spec2d = spec_p.reshape(N_PAD, 1)
    skip0_tab = l0_Wskip[jnp.arange(N_SPECIES), jnp.arange(N_SPECIES), :]
    W_all = jnp.transpose(l1_Wskip, (1, 0, 2)).reshape(
        HIDDEN, N_SPECIES * HIDDEN)
    ae2d = atom_energies.reshape(N_SPECIES, 1).astype(f32)
    zeros0 = jnp.zeros((N_PAD, F0), f32)
    zeros1 = jnp.zeros((N_PAD, F1), f32)

    sq = _geom_call(posx, posy, posz, snd_p, rcv_p)
    rad0, rad1 = _radial_call(sq.reshape(EP, 1),
                              l0_R0, l0_R1, l0_R2, l1_R0, l1_R1, l1_R2)
    node0 = _node0_call(spec2d, l0_W1)
    part0 = _msgpass_call(node0, snd_p, rcv2d, rad0, zeros0, F0,
                          edge_split=True).reshape(2, N_PAD, F0)
    feat1, node1 = _comb0_call(part0, spec2d, skip0_tab, l0_W2, l1_W1)
    agg1 = _msgpass_call(node1.reshape(2 * N_PAD, F1), snd_p, rcv2d,
                         rad1.reshape(2 * EP, F1), zeros1, F1,
                         edge_split=False, G=256).reshape(2, N_PAD, F1)
    e = _comb1_call(agg1, feat1, spec2d, W_all, l1_W2, W_ro, ae2d)
    return e[:N]


# speedup vs baseline: 3.3620x; 1.6448x over previous
import functools

import jax
import jax.numpy as jnp
import numpy as np
from jax import lax
from jax.experimental import pallas as pl
from jax.experimental.pallas import tpu as pltpu
from jax.experimental.pallas import tpu_sc as plsc

N_SPECIES = 8
HIDDEN = 64
RB = 8
RM = 64
CUTOFF = 5.0
AVG_N = 16.0
SCALE = 1.0
SHIFT = 0.0
N = 50000
E = 800000

N_PAD = 50176
EP = 819200
TRASH = N

NC = 2
NS = 16
G = 1024
GC = G // 128

F0 = 16
F1 = 32

BN = 1792
BE = 2048


def _silu(x):
    return x * (1.0 / (1.0 + jnp.exp(-x)))


def _sc_mesh():
    return plsc.VectorSubcoreMesh(core_axis_name="c", subcore_axis_name="s")


def _geom_call(posx, posy, posz, senders, receivers):
    ET = EP // (NC * NS)
    NG = ET // G

    def body(px_hbm, py_hbm, pz_hbm, snd_hbm, rcv_hbm, sq_hbm,
             sidx, ridx, xs, ys, zs, xr, yr, zr, sqv, sem):
        c = lax.axis_index("c")
        s = lax.axis_index("s")
        wid = s * NC + c
        base0 = wid * ET

        def group(g, carry):
            base = base0 + g * G
            pltpu.sync_copy(snd_hbm.at[pl.ds(base, G)], sidx)
            pltpu.sync_copy(rcv_hbm.at[pl.ds(base, G)], ridx)
            cps = [
                pltpu.async_copy(px_hbm.at[sidx], xs, sem),
                pltpu.async_copy(py_hbm.at[sidx], ys, sem),
                pltpu.async_copy(pz_hbm.at[sidx], zs, sem),
                pltpu.async_copy(px_hbm.at[ridx], xr, sem),
                pltpu.async_copy(py_hbm.at[ridx], yr, sem),
                pltpu.async_copy(pz_hbm.at[ridx], zr, sem),
            ]
            for cp in cps:
                cp.wait()

            def sub(i, carry2):
                sl = pl.ds(i * 16, 16)
                dx = xs[sl] - xr[sl]
                dy = ys[sl] - yr[sl]
                dz = zs[sl] - zr[sl]
                sqv[sl] = dx * dx + dy * dy + dz * dz
                return carry2

            lax.fori_loop(0, G // 16, sub, 0)
            pltpu.sync_copy(sqv, sq_hbm.at[pl.ds(base, G)])
            return carry

        lax.fori_loop(0, NG, group, 0)

    k = pl.kernel(
        body,
        out_type=jax.ShapeDtypeStruct((EP,), jnp.float32),
        mesh=_sc_mesh(),
        compiler_params=pltpu.CompilerParams(use_tc_tiling_on_sc=False),
        scratch_types=[
            pltpu.VMEM((G,), jnp.int32),
            pltpu.VMEM((G,), jnp.int32),
            pltpu.VMEM((G,), jnp.float32),
            pltpu.VMEM((G,), jnp.float32),
            pltpu.VMEM((G,), jnp.float32),
            pltpu.VMEM((G,), jnp.float32),
            pltpu.VMEM((G,), jnp.float32),
            pltpu.VMEM((G,), jnp.float32),
            pltpu.VMEM((G,), jnp.float32),
            pltpu.SemaphoreType.DMA,
        ],
    )
    return k(posx, posy, posz, senders, receivers)


def _msgpass_call(node_tab, senders, recv2d, rad, zeros_nf, width, edge_split,
                  G=G):
    GC = G // 128
    ET = (EP // 2 if edge_split else EP) // NS
    SUPER = 1024
    NSUB = SUPER // G
    NG = ET // SUPER
    ROWS_T = N_PAD // NS
    WCH = max(w for w in (784, 448, 224, 112, 56)
              if w <= G and ROWS_T % w == 0)
    NWC = ROWS_T // WCH

    def body(node_hbm, snd_hbm, rcv_hbm, rad_hbm, zero_hbm, out_hbm,
             sidx, ridx2, rows, radv, shared, sem):
        c = lax.axis_index("c")
        s = lax.axis_index("s")

        for k2 in range(NWC):
            roff = s * ROWS_T + k2 * WCH
            pltpu.sync_copy(zero_hbm.at[pl.ds(roff, WCH)],
                            shared.at[pl.ds(roff, WCH)])
        plsc.subcore_barrier()

        if edge_split:
            ebase = c * (EP // 2) + s * ET
            rad_base0 = ebase
        else:
            ebase = s * ET
            rad_base0 = c * EP + s * ET

        def group(g, carry):
            base = ebase + g * SUPER
            rbase = rad_base0 + g * SUPER
            pltpu.sync_copy(snd_hbm.at[pl.ds(base, SUPER)], sidx)
            pltpu.sync_copy(
                rcv_hbm.at[pl.ds(pl.multiple_of(base // 128, 8), SUPER // 128)],
                ridx2)
            if not edge_split:
                off = c * N_PAD

                def addoff(i, carry2):
                    sidx[pl.ds(i * 16, 16)] = sidx[pl.ds(i * 16, 16)] + off
                    return carry2

                lax.fori_loop(0, SUPER // 16, addoff, 0)
            for sub in range(NSUB):
                gcp = pltpu.async_copy(
                    node_hbm.at[sidx.at[pl.ds(sub * G, G)]], rows, sem)
                pltpu.sync_copy(rad_hbm.at[pl.ds(rbase + sub * G, G)], radv)
                gcp.wait()

                def mul(i, carry2):
                    for j in range(width // 16):
                        sl = pl.ds(j * 16, 16)
                        rows[i, sl] = rows[i, sl] * radv[i, sl]
                    return carry2

                lax.fori_loop(0, G, mul, 0)
                for j in range(GC):
                    pltpu.sync_copy(rows.at[pl.ds(j * 128, 128)],
                                    shared.at[ridx2.at[sub * GC + j]],
                                    add=True)
            return carry

        lax.fori_loop(0, NG, group, 0)
        plsc.subcore_barrier()
        for k2 in range(NWC):
            roff = s * ROWS_T + k2 * WCH
            pltpu.sync_copy(shared.at[pl.ds(roff, WCH)],
                            rows.at[pl.ds(0, WCH)])
            pltpu.sync_copy(rows.at[pl.ds(0, WCH)],
                            out_hbm.at[pl.ds(c * N_PAD + roff, WCH)])

    k = pl.kernel(
        body,
        out_type=jax.ShapeDtypeStruct((2 * N_PAD, width), jnp.float32),
        mesh=_sc_mesh(),
        compiler_params=pltpu.CompilerParams(use_tc_tiling_on_sc=False),
        scratch_types=[
            pltpu.VMEM((SUPER,), jnp.int32),
            pltpu.VMEM((SUPER // 128, 128), jnp.int32),
            pltpu.VMEM((G, width), jnp.float32),
            pltpu.VMEM((G, width), jnp.float32),
            pltpu.VMEM_SHARED((N_PAD, width), jnp.float32),
            pltpu.SemaphoreType.DMA,
        ],
    )
    return k(node_tab, senders, recv2d, rad, zeros_nf)


def _radial_body(sq_ref, r0cat, r1diag, r2diag, rad0_ref, rad1_ref):
    for sb in range(8):
        x = sq_ref[sb:sb + 1, :]
        iszero = x == 0.0
        rn = jnp.where(iszero, 0.0, jnp.sqrt(jnp.where(iszero, 1.0, x)))
        xs = jnp.where(rn == 0.0, 1.0, rn)
        xb = jnp.broadcast_to(xs, (RB, BE))
        wm = (lax.broadcasted_iota(jnp.int32, (RB, BE), 0) + 1
              ).astype(jnp.float32) * np.pi
        bes = (2.0 / CUTOFF) * jnp.where(
            rn == 0.0, wm / CUTOFF, jnp.sin(wm * xb / CUTOFF) / xb)
        xh = rn * (1.0 / CUTOFF)
        x2 = xh * xh
        cut = (1.0 - 6.0 * x2 + 8.0 * x2 * xh - 3.0 * x2 * x2)
        cut = cut * jnp.where(xh < 1.0, 1.0, 0.0)
        rbT = bes * cut
        h1 = _silu(lax.dot_general(
            rbT, r0cat[...], (((0,), (0,)), ((), ())),
            preferred_element_type=jnp.float32))
        h2 = _silu(jnp.dot(h1.astype(jnp.bfloat16), r1diag[...],
                           preferred_element_type=jnp.float32))
        rad = jnp.dot(h2.astype(jnp.bfloat16), r2diag[...],
                      preferred_element_type=jnp.float32)
        sl = pl.ds(sb * BE, BE)
        rad0_ref[sl, :] = rad[:, :F0]
        rad1_ref[0, sl, :] = rad[:, F0:F0 + F1]
        rad1_ref[1, sl, :] = rad[:, F0 + F1:]


def _radial_call(sq2d, R0cat, R1diag, R2diag):
    grid = (EP // (8 * BE),)
    return pl.pallas_call(
        _radial_body,
        grid=grid,
        in_specs=[
            pl.BlockSpec((8, BE), lambda i: (i, 0)),
            pl.BlockSpec((RB, 2 * RM), lambda i: (0, 0)),
            pl.BlockSpec((2 * RM, 2 * RM), lambda i: (0, 0)),
            pl.BlockSpec((2 * RM, F0 + 2 * F1), lambda i: (0, 0)),
        ],
        out_specs=[
            pl.BlockSpec((8 * BE, F0), lambda i: (i, 0)),
            pl.BlockSpec((2, 8 * BE, F1), lambda i: (0, i, 0)),
        ],
        out_shape=[
            jax.ShapeDtypeStruct((EP, F0), jnp.float32),
            jax.ShapeDtypeStruct((2, EP, F1), jnp.float32),
        ],
    )(sq2d, R0cat, R1diag, R2diag)


def _node0_body(spec_ref, w1_ref, out_ref):
    spec = spec_ref[...]
    oh = (spec == lax.broadcasted_iota(jnp.int32, (BN, N_SPECIES), 1))
    oh = oh.astype(jnp.float32)
    n0 = jnp.dot(oh, w1_ref[...], preferred_element_type=jnp.float32)
    out_ref[...] = jnp.pad(n0, ((0, 0), (0, F0 - N_SPECIES)))


def _node0_call(spec2d, W1_0):
    return pl.pallas_call(
        _node0_body,
        grid=(N_PAD // BN,),
        in_specs=[
            pl.BlockSpec((BN, 1), lambda i: (i, 0)),
            pl.BlockSpec((N_SPECIES, N_SPECIES), lambda i: (0, 0)),
        ],
        out_specs=pl.BlockSpec((BN, F0), lambda i: (i, 0)),
        out_shape=jax.ShapeDtypeStruct((N_PAD, F0), jnp.float32),
    )(spec2d, W1_0)


def _comb0_body(part_ref, spec_ref, skiptab_ref, w2_ref, w1n_ref,
                feat1_ref, node1_ref):
    p = part_ref[0] + part_ref[1]
    agg = p[:, :N_SPECIES] * (1.0 / np.sqrt(AVG_N))
    spec = spec_ref[...]
    oh = (spec == lax.broadcasted_iota(jnp.int32, (BN, N_SPECIES), 1))
    oh = oh.astype(jnp.float32)
    skip = jnp.dot(oh, skiptab_ref[...], preferred_element_type=jnp.float32)
    f1 = _silu(jnp.dot(agg, w2_ref[...],
                       preferred_element_type=jnp.float32) + skip)
    feat1_ref[...] = f1
    n1 = jnp.dot(f1, w1n_ref[...], preferred_element_type=jnp.float32)
    node1_ref[0] = n1[:, :F1]
    node1_ref[1] = n1[:, F1:]


def _comb0_call(part0, spec2d, skip0_tab, W2_0, W1_1):
    return pl.pallas_call(
        _comb0_body,
        grid=(N_PAD // BN,),
        in_specs=[
            pl.BlockSpec((2, BN, F0), lambda i: (0, i, 0)),
            pl.BlockSpec((BN, 1), lambda i: (i, 0)),
            pl.BlockSpec((N_SPECIES, HIDDEN), lambda i: (0, 0)),
            pl.BlockSpec((N_SPECIES, HIDDEN), lambda i: (0, 0)),
            pl.BlockSpec((HIDDEN, HIDDEN), lambda i: (0, 0)),
        ],
        out_specs=[
            pl.BlockSpec((BN, HIDDEN), lambda i: (i, 0)),
            pl.BlockSpec((2, BN, F1), lambda i: (0, i, 0)),
        ],
        out_shape=[
            jax.ShapeDtypeStruct((N_PAD, HIDDEN), jnp.float32),
            jax.ShapeDtypeStruct((2, N_PAD, F1), jnp.float32),
        ],
    )(part0, spec2d, skip0_tab, W2_0, W1_1)


def _comb1_body(agg_ref, feat1_ref, spec_ref, wall_ref, w2_ref, wro_ref,
                ae_ref, out_ref):
    agg = jnp.concatenate([agg_ref[0], agg_ref[1]], axis=1)
    agg = agg * (1.0 / np.sqrt(AVG_N))
    f1 = feat1_ref[...]
    hs = jnp.dot(f1, wall_ref[...], preferred_element_type=jnp.float32)
    spec = spec_ref[...]
    skip = jnp.zeros((BN, HIDDEN), jnp.float32)
    for s in range(N_SPECIES):
        skip = skip + jnp.where(spec == s,
                                hs[:, s * HIDDEN:(s + 1) * HIDDEN], 0.0)
    f2 = _silu(jnp.dot(agg, w2_ref[...],
                       preferred_element_type=jnp.float32) + skip)
    e = jnp.dot(f2, wro_ref[...], preferred_element_type=jnp.float32)
    e = e * SCALE + SHIFT
    oh = (spec == lax.broadcasted_iota(jnp.int32, (BN, N_SPECIES), 1))
    e = e + jnp.dot(oh.astype(jnp.float32), ae_ref[...],
                    preferred_element_type=jnp.float32)
    out_ref[...] = e


def _comb1_call(agg1, feat1, spec2d, W_all, W2_1, W_ro, ae2d):
    return pl.pallas_call(
        _comb1_body,
        grid=(N_PAD // BN,),
        in_specs=[
            pl.BlockSpec((2, BN, F1), lambda i: (0, i, 0)),
            pl.BlockSpec((BN, HIDDEN), lambda i: (i, 0)),
            pl.BlockSpec((BN, 1), lambda i: (i, 0)),
            pl.BlockSpec((HIDDEN, N_SPECIES * HIDDEN), lambda i: (0, 0)),
            pl.BlockSpec((HIDDEN, HIDDEN), lambda i: (0, 0)),
            pl.BlockSpec((HIDDEN, 1), lambda i: (0, 0)),
            pl.BlockSpec((N_SPECIES, 1), lambda i: (0, 0)),
        ],
        out_specs=pl.BlockSpec((BN, 1), lambda i: (i, 0)),
        out_shape=jax.ShapeDtypeStruct((N_PAD, 1), jnp.float32),
    )(agg1, feat1, spec2d, W_all, W2_1, W_ro, ae2d)


def kernel(positions, species, senders, receivers,
           l0_W1, l0_R0, l0_R1, l0_R2, l0_W2, l0_Wskip,
           l1_W1, l1_R0, l1_R1, l1_R2, l1_W2, l1_Wskip,
           W_ro, atom_energies):
    f32 = jnp.float32
    posx = jnp.zeros((N_PAD,), f32).at[:N].set(positions[:, 0].astype(f32))
    posy = jnp.zeros((N_PAD,), f32).at[:N].set(positions[:, 1].astype(f32))
    posz = jnp.zeros((N_PAD,), f32).at[:N].set(positions[:, 2].astype(f32))
    spec_p = jnp.zeros((N_PAD,), jnp.int32).at[:N].set(
        species.astype(jnp.int32))
    snd_p = jnp.full((EP,), TRASH, jnp.int32).at[:E].set(
        senders.astype(jnp.int32))
    rcv_p = jnp.full((EP,), TRASH, jnp.int32).at[:E].set(
        receivers.astype(jnp.int32))
    rcv2d = rcv_p.reshape(EP // 128, 128)
    spec2d = spec_p.reshape(N_PAD, 1)
    skip0_tab = l0_Wskip[jnp.arange(N_SPECIES), jnp.arange(N_SPECIES), :]
    W_all = jnp.transpose(l1_Wskip, (1, 0, 2)).reshape(
        HIDDEN, N_SPECIES * HIDDEN)
    ae2d = atom_energies.reshape(N_SPECIES, 1).astype(f32)
    zeros0 = jnp.zeros((N_PAD, F0), f32)
    zeros1 = jnp.zeros((N_PAD, F1), f32)

    R0cat = jnp.concatenate([l0_R0, l1_R0], axis=1)
    R1diag = (jnp.zeros((2 * RM, 2 * RM), f32)
              .at[:RM, :RM].set(l0_R1)
              .at[RM:, RM:].set(l1_R1).astype(jnp.bfloat16))
    R2diag = (jnp.zeros((2 * RM, F0 + 2 * F1), f32)
              .at[:RM, :RB].set(l0_R2)
              .at[RM:, F0:].set(l1_R2).astype(jnp.bfloat16))
    sq = _geom_call(posx, posy, posz, snd_p, rcv_p)
    rad0, rad1 = _radial_call(sq.reshape(EP // BE, BE), R0cat, R1diag, R2diag)
    node0 = _node0_call(spec2d, l0_W1)
    part0 = _msgpass_call(node0, snd_p, rcv2d, rad0, zeros0, F0,
                          edge_split=True).reshape(2, N_PAD, F0)
    feat1, node1 = _comb0_call(part0, spec2d, skip0_tab, l0_W2, l1_W1)
    agg1 = _msgpass_call(node1.reshape(2 * N_PAD, F1), snd_p, rcv2d,
                         rad1.reshape(2 * EP, F1), zeros1, F1,
                         edge_split=False, G=256).reshape(2, N_PAD, F1)
    e = _comb1_call(agg1, feat1, spec2d, W_all, l1_W2, W_ro, ae2d)
    return e[:N]


# speedup vs baseline: 3.5070x; 1.0431x over previous
import functools

import jax
import jax.numpy as jnp
import numpy as np
from jax import lax
from jax.experimental import pallas as pl
from jax.experimental.pallas import tpu as pltpu
from jax.experimental.pallas import tpu_sc as plsc

N_SPECIES = 8
HIDDEN = 64
RB = 8
RM = 64
CUTOFF = 5.0
AVG_N = 16.0
SCALE = 1.0
SHIFT = 0.0
N = 50000
E = 800000

N_PAD = 50176
EP = 819200
TRASH = N

NC = 2
NS = 16
G = 1024
GC = G // 128

F0 = 16
F1 = 32

BN = 1792
BE = 2048


def _silu(x):
    return x * (1.0 / (1.0 + jnp.exp(-x)))


def _sc_mesh():
    return plsc.VectorSubcoreMesh(core_axis_name="c", subcore_axis_name="s")


def _geom_call(posx, posy, posz, senders, receivers):
    ET = EP // (NC * NS)
    NG = ET // G

    def body(px_hbm, py_hbm, pz_hbm, snd_hbm, rcv_hbm, sq_hbm,
             sidx, ridx, xs, ys, zs, xr, yr, zr, sqv, sem):
        c = lax.axis_index("c")
        s = lax.axis_index("s")
        wid = s * NC + c
        base0 = wid * ET

        def group(g, carry):
            base = base0 + g * G
            pltpu.sync_copy(snd_hbm.at[pl.ds(base, G)], sidx)
            pltpu.sync_copy(rcv_hbm.at[pl.ds(base, G)], ridx)
            cps = [
                pltpu.async_copy(px_hbm.at[sidx], xs, sem),
                pltpu.async_copy(py_hbm.at[sidx], ys, sem),
                pltpu.async_copy(pz_hbm.at[sidx], zs, sem),
                pltpu.async_copy(px_hbm.at[ridx], xr, sem),
                pltpu.async_copy(py_hbm.at[ridx], yr, sem),
                pltpu.async_copy(pz_hbm.at[ridx], zr, sem),
            ]
            for cp in cps:
                cp.wait()

            @plsc.parallel_loop(0, G // 16, unroll=8)
            def _sub(i):
                sl = pl.ds(i * 16, 16)
                dx = xs[sl] - xr[sl]
                dy = ys[sl] - yr[sl]
                dz = zs[sl] - zr[sl]
                sqv[sl] = dx * dx + dy * dy + dz * dz
            pltpu.sync_copy(sqv, sq_hbm.at[pl.ds(base, G)])
            return carry

        lax.fori_loop(0, NG, group, 0)

    k = pl.kernel(
        body,
        out_type=jax.ShapeDtypeStruct((EP,), jnp.float32),
        mesh=_sc_mesh(),
        compiler_params=pltpu.CompilerParams(use_tc_tiling_on_sc=False),
        scratch_types=[
            pltpu.VMEM((G,), jnp.int32),
            pltpu.VMEM((G,), jnp.int32),
            pltpu.VMEM((G,), jnp.float32),
            pltpu.VMEM((G,), jnp.float32),
            pltpu.VMEM((G,), jnp.float32),
            pltpu.VMEM((G,), jnp.float32),
            pltpu.VMEM((G,), jnp.float32),
            pltpu.VMEM((G,), jnp.float32),
            pltpu.VMEM((G,), jnp.float32),
            pltpu.SemaphoreType.DMA,
        ],
    )
    return k(posx, posy, posz, senders, receivers)


def _msgpass_call(node_tab, senders, recv2d, rad, zeros_nf, width, edge_split,
                  G=G):
    GC = G // 128
    ET = (EP // 2 if edge_split else EP) // NS
    SUPER = 1024
    NSUB = SUPER // G
    NG = ET // SUPER
    ROWS_T = N_PAD // NS
    WCH = max(w for w in (784, 448, 224, 112, 56)
              if w <= G and ROWS_T % w == 0)
    NWC = ROWS_T // WCH

    def body(node_hbm, snd_hbm, rcv_hbm, rad_hbm, zero_hbm, out_hbm,
             sidx, ridx2, rows, radv, shared, sem):
        c = lax.axis_index("c")
        s = lax.axis_index("s")

        for k2 in range(NWC):
            roff = s * ROWS_T + k2 * WCH
            pltpu.sync_copy(zero_hbm.at[pl.ds(roff, WCH)],
                            shared.at[pl.ds(roff, WCH)])
        plsc.subcore_barrier()

        if edge_split:
            ebase = c * (EP // 2) + s * ET
            rad_base0 = ebase
        else:
            ebase = s * ET
            rad_base0 = c * EP + s * ET

        def group(g, carry):
            base = ebase + g * SUPER
            rbase = rad_base0 + g * SUPER
            pltpu.sync_copy(snd_hbm.at[pl.ds(base, SUPER)], sidx)
            pltpu.sync_copy(
                rcv_hbm.at[pl.ds(pl.multiple_of(base // 128, 8), SUPER // 128)],
                ridx2)
            if not edge_split:
                off = c * N_PAD

                @plsc.parallel_loop(0, SUPER // 16, unroll=8)
                def _addoff(i):
                    sidx[pl.ds(i * 16, 16)] = sidx[pl.ds(i * 16, 16)] + off
            for sub in range(NSUB):
                gcp = pltpu.async_copy(
                    node_hbm.at[sidx.at[pl.ds(sub * G, G)]], rows, sem)
                pltpu.sync_copy(rad_hbm.at[pl.ds(rbase + sub * G, G)], radv)
                gcp.wait()

                @plsc.parallel_loop(0, G, unroll=8)
                def _mul(i):
                    for j in range(width // 16):
                        sl = pl.ds(j * 16, 16)
                        rows[i, sl] = rows[i, sl] * radv[i, sl]
                for j in range(GC):
                    pltpu.sync_copy(rows.at[pl.ds(j * 128, 128)],
                                    shared.at[ridx2.at[sub * GC + j]],
                                    add=True)
            return carry

        lax.fori_loop(0, NG, group, 0)
        plsc.subcore_barrier()
        for k2 in range(NWC):
            roff = s * ROWS_T + k2 * WCH
            pltpu.sync_copy(shared.at[pl.ds(roff, WCH)],
                            rows.at[pl.ds(0, WCH)])
            pltpu.sync_copy(rows.at[pl.ds(0, WCH)],
                            out_hbm.at[pl.ds(c * N_PAD + roff, WCH)])

    k = pl.kernel(
        body,
        out_type=jax.ShapeDtypeStruct((2 * N_PAD, width), jnp.float32),
        mesh=_sc_mesh(),
        compiler_params=pltpu.CompilerParams(use_tc_tiling_on_sc=False),
        scratch_types=[
            pltpu.VMEM((SUPER,), jnp.int32),
            pltpu.VMEM((SUPER // 128, 128), jnp.int32),
            pltpu.VMEM((G, width), jnp.float32),
            pltpu.VMEM((G, width), jnp.float32),
            pltpu.VMEM_SHARED((N_PAD, width), jnp.float32),
            pltpu.SemaphoreType.DMA,
        ],
    )
    return k(node_tab, senders, recv2d, rad, zeros_nf)


def _radial_body(sq_ref, r0cat, r1diag, r2diag, rad0_ref, rad1_ref):
    for sb in range(8):
        x = sq_ref[sb:sb + 1, :]
        iszero = x == 0.0
        rn = jnp.where(iszero, 0.0, jnp.sqrt(jnp.where(iszero, 1.0, x)))
        xs = jnp.where(rn == 0.0, 1.0, rn)
        xb = jnp.broadcast_to(xs, (RB, BE))
        wm = (lax.broadcasted_iota(jnp.int32, (RB, BE), 0) + 1
              ).astype(jnp.float32) * np.pi
        bes = (2.0 / CUTOFF) * jnp.where(
            rn == 0.0, wm / CUTOFF, jnp.sin(wm * xb / CUTOFF) / xb)
        xh = rn * (1.0 / CUTOFF)
        x2 = xh * xh
        cut = (1.0 - 6.0 * x2 + 8.0 * x2 * xh - 3.0 * x2 * x2)
        cut = cut * jnp.where(xh < 1.0, 1.0, 0.0)
        rbT = bes * cut
        h1 = _silu(lax.dot_general(
            rbT, r0cat[...], (((0,), (0,)), ((), ())),
            preferred_element_type=jnp.float32))
        h2 = _silu(jnp.dot(h1.astype(jnp.bfloat16), r1diag[...],
                           preferred_element_type=jnp.float32))
        rad = jnp.dot(h2.astype(jnp.bfloat16), r2diag[...],
                      preferred_element_type=jnp.float32)
        sl = pl.ds(sb * BE, BE)
        rad0_ref[sl, :] = rad[:, :F0]
        rad1_ref[0, sl, :] = rad[:, F0:F0 + F1]
        rad1_ref[1, sl, :] = rad[:, F0 + F1:]


def _radial_call(sq2d, R0cat, R1diag, R2diag):
    grid = (EP // (8 * BE),)
    return pl.pallas_call(
        _radial_body,
        grid=grid,
        in_specs=[
            pl.BlockSpec((8, BE), lambda i: (i, 0)),
            pl.BlockSpec((RB, 2 * RM), lambda i: (0, 0)),
            pl.BlockSpec((2 * RM, 2 * RM), lambda i: (0, 0)),
            pl.BlockSpec((2 * RM, F0 + 2 * F1), lambda i: (0, 0)),
        ],
        out_specs=[
            pl.BlockSpec((8 * BE, F0), lambda i: (i, 0)),
            pl.BlockSpec((2, 8 * BE, F1), lambda i: (0, i, 0)),
        ],
        out_shape=[
            jax.ShapeDtypeStruct((EP, F0), jnp.float32),
            jax.ShapeDtypeStruct((2, EP, F1), jnp.float32),
        ],
    )(sq2d, R0cat, R1diag, R2diag)


def _node0_body(spec_ref, w1_ref, out_ref):
    spec = spec_ref[...]
    oh = (spec == lax.broadcasted_iota(jnp.int32, (BN, N_SPECIES), 1))
    oh = oh.astype(jnp.float32)
    n0 = jnp.dot(oh, w1_ref[...], preferred_element_type=jnp.float32)
    out_ref[...] = jnp.pad(n0, ((0, 0), (0, F0 - N_SPECIES)))


def _node0_call(spec2d, W1_0):
    return pl.pallas_call(
        _node0_body,
        grid=(N_PAD // BN,),
        in_specs=[
            pl.BlockSpec((BN, 1), lambda i: (i, 0)),
            pl.BlockSpec((N_SPECIES, N_SPECIES), lambda i: (0, 0)),
        ],
        out_specs=pl.BlockSpec((BN, F0), lambda i: (i, 0)),
        out_shape=jax.ShapeDtypeStruct((N_PAD, F0), jnp.float32),
    )(spec2d, W1_0)


def _comb0_body(part_ref, spec_ref, skiptab_ref, w2_ref, w1n_ref,
                feat1_ref, node1_ref):
    p = part_ref[0] + part_ref[1]
    agg = p[:, :N_SPECIES] * (1.0 / np.sqrt(AVG_N))
    spec = spec_ref[...]
    oh = (spec == lax.broadcasted_iota(jnp.int32, (BN, N_SPECIES), 1))
    oh = oh.astype(jnp.float32)
    skip = jnp.dot(oh, skiptab_ref[...], preferred_element_type=jnp.float32)
    f1 = _silu(jnp.dot(agg, w2_ref[...],
                       preferred_element_type=jnp.float32) + skip)
    feat1_ref[...] = f1
    n1 = jnp.dot(f1, w1n_ref[...], preferred_element_type=jnp.float32)
    node1_ref[0] = n1[:, :F1]
    node1_ref[1] = n1[:, F1:]


def _comb0_call(part0, spec2d, skip0_tab, W2_0, W1_1):
    return pl.pallas_call(
        _comb0_body,
        grid=(N_PAD // BN,),
        in_specs=[
            pl.BlockSpec((2, BN, F0), lambda i: (0, i, 0)),
            pl.BlockSpec((BN, 1), lambda i: (i, 0)),
            pl.BlockSpec((N_SPECIES, HIDDEN), lambda i: (0, 0)),
            pl.BlockSpec((N_SPECIES, HIDDEN), lambda i: (0, 0)),
            pl.BlockSpec((HIDDEN, HIDDEN), lambda i: (0, 0)),
        ],
        out_specs=[
            pl.BlockSpec((BN, HIDDEN), lambda i: (i, 0)),
            pl.BlockSpec((2, BN, F1), lambda i: (0, i, 0)),
        ],
        out_shape=[
            jax.ShapeDtypeStruct((N_PAD, HIDDEN), jnp.float32),
            jax.ShapeDtypeStruct((2, N_PAD, F1), jnp.float32),
        ],
    )(part0, spec2d, skip0_tab, W2_0, W1_1)


def _comb1_body(agg_ref, feat1_ref, spec_ref, wall_ref, w2_ref, wro_ref,
                ae_ref, out_ref):
    agg = jnp.concatenate([agg_ref[0], agg_ref[1]], axis=1)
    agg = agg * (1.0 / np.sqrt(AVG_N))
    f1 = feat1_ref[...]
    hs = jnp.dot(f1, wall_ref[...], preferred_element_type=jnp.float32)
    spec = spec_ref[...]
    skip = jnp.zeros((BN, HIDDEN), jnp.float32)
    for s in range(N_SPECIES):
        skip = skip + jnp.where(spec == s,
                                hs[:, s * HIDDEN:(s + 1) * HIDDEN], 0.0)
    f2 = _silu(jnp.dot(agg, w2_ref[...],
                       preferred_element_type=jnp.float32) + skip)
    e = jnp.dot(f2, wro_ref[...], preferred_element_type=jnp.float32)
    e = e * SCALE + SHIFT
    oh = (spec == lax.broadcasted_iota(jnp.int32, (BN, N_SPECIES), 1))
    e = e + jnp.dot(oh.astype(jnp.float32), ae_ref[...],
                    preferred_element_type=jnp.float32)
    out_ref[...] = e


def _comb1_call(agg1, feat1, spec2d, W_all, W2_1, W_ro, ae2d):
    return pl.pallas_call(
        _comb1_body,
        grid=(N_PAD // BN,),
        in_specs=[
            pl.BlockSpec((2, BN, F1), lambda i: (0, i, 0)),
            pl.BlockSpec((BN, HIDDEN), lambda i: (i, 0)),
            pl.BlockSpec((BN, 1), lambda i: (i, 0)),
            pl.BlockSpec((HIDDEN, N_SPECIES * HIDDEN), lambda i: (0, 0)),
            pl.BlockSpec((HIDDEN, HIDDEN), lambda i: (0, 0)),
            pl.BlockSpec((HIDDEN, 1), lambda i: (0, 0)),
            pl.BlockSpec((N_SPECIES, 1), lambda i: (0, 0)),
        ],
        out_specs=pl.BlockSpec((BN, 1), lambda i: (i, 0)),
        out_shape=jax.ShapeDtypeStruct((N_PAD, 1), jnp.float32),
    )(agg1, feat1, spec2d, W_all, W2_1, W_ro, ae2d)


def kernel(positions, species, senders, receivers,
           l0_W1, l0_R0, l0_R1, l0_R2, l0_W2, l0_Wskip,
           l1_W1, l1_R0, l1_R1, l1_R2, l1_W2, l1_Wskip,
           W_ro, atom_energies):
    f32 = jnp.float32
    posx = jnp.zeros((N_PAD,), f32).at[:N].set(positions[:, 0].astype(f32))
    posy = jnp.zeros((N_PAD,), f32).at[:N].set(positions[:, 1].astype(f32))
    posz = jnp.zeros((N_PAD,), f32).at[:N].set(positions[:, 2].astype(f32))
    spec_p = jnp.zeros((N_PAD,), jnp.int32).at[:N].set(
        species.astype(jnp.int32))
    snd_p = jnp.full((EP,), TRASH, jnp.int32).at[:E].set(
        senders.astype(jnp.int32))
    rcv_p = jnp.full((EP,), TRASH, jnp.int32).at[:E].set(
        receivers.astype(jnp.int32))
    rcv2d = rcv_p.reshape(EP // 128, 128)
    spec2d = spec_p.reshape(N_PAD, 1)
    skip0_tab = l0_Wskip[jnp.arange(N_SPECIES), jnp.arange(N_SPECIES), :]
    W_all = jnp.transpose(l1_Wskip, (1, 0, 2)).reshape(
        HIDDEN, N_SPECIES * HIDDEN)
    ae2d = atom_energies.reshape(N_SPECIES, 1).astype(f32)
    zeros0 = jnp.zeros((N_PAD, F0), f32)
    zeros1 = jnp.zeros((N_PAD, F1), f32)

    R0cat = jnp.concatenate([l0_R0, l1_R0], axis=1)
    R1diag = (jnp.zeros((2 * RM, 2 * RM), f32)
              .at[:RM, :RM].set(l0_R1)
              .at[RM:, RM:].set(l1_R1).astype(jnp.bfloat16))
    R2diag = (jnp.zeros((2 * RM, F0 + 2 * F1), f32)
              .at[:RM, :RB].set(l0_R2)
              .at[RM:, F0:].set(l1_R2).astype(jnp.bfloat16))
    sq = _geom_call(posx, posy, posz, snd_p, rcv_p)
    rad0, rad1 = _radial_call(sq.reshape(EP // BE, BE), R0cat, R1diag, R2diag)
    node0 = _node0_call(spec2d, l0_W1)
    part0 = _msgpass_call(node0, snd_p, rcv2d, rad0, zeros0, F0,
                          edge_split=True).reshape(2, N_PAD, F0)
    feat1, node1 = _comb0_call(part0, spec2d, skip0_tab, l0_W2, l1_W1)
    agg1 = _msgpass_call(node1.reshape(2 * N_PAD, F1), snd_p, rcv2d,
                         rad1.reshape(2 * EP, F1), zeros1, F1,
                         edge_split=False, G=256).reshape(2, N_PAD, F1)
    e = _comb1_call(agg1, feat1, spec2d, W_all, l1_W2, W_ro, ae2d)
    return e[:N]


# speedup vs baseline: 3.9384x; 1.1230x over previous
import functools

import jax
import jax.numpy as jnp
import numpy as np
from jax import lax
from jax.experimental import pallas as pl
from jax.experimental.pallas import tpu as pltpu
from jax.experimental.pallas import tpu_sc as plsc

N_SPECIES = 8
HIDDEN = 64
RB = 8
RM = 64
CUTOFF = 5.0
AVG_N = 16.0
SCALE = 1.0
SHIFT = 0.0
N = 50000
E = 800000

N_PAD = 50176
EP = 819200
TRASH = N

NC = 2
NS = 16
G = 1024
GC = G // 128

F0 = 16
F1 = 32

BN = 1792
BE = 2048


def _silu(x):
    return x * (1.0 / (1.0 + jnp.exp(-x)))


def _sc_mesh():
    return plsc.VectorSubcoreMesh(core_axis_name="c", subcore_axis_name="s")


def _geom_call(posx, posy, posz, senders, receivers):
    ET = EP // (NC * NS)
    NG = ET // G

    def body(px_hbm, py_hbm, pz_hbm, snd_hbm, rcv_hbm, sq_hbm,
             sidx, ridx, xs, ys, zs, xr, yr, zr, sqv, sem):
        c = lax.axis_index("c")
        s = lax.axis_index("s")
        wid = s * NC + c
        base0 = wid * ET

        def group(g, carry):
            base = base0 + g * G
            pltpu.sync_copy(snd_hbm.at[pl.ds(base, G)], sidx)
            pltpu.sync_copy(rcv_hbm.at[pl.ds(base, G)], ridx)
            cps = [
                pltpu.async_copy(px_hbm.at[sidx], xs, sem),
                pltpu.async_copy(py_hbm.at[sidx], ys, sem),
                pltpu.async_copy(pz_hbm.at[sidx], zs, sem),
                pltpu.async_copy(px_hbm.at[ridx], xr, sem),
                pltpu.async_copy(py_hbm.at[ridx], yr, sem),
                pltpu.async_copy(pz_hbm.at[ridx], zr, sem),
            ]
            for cp in cps:
                cp.wait()

            @plsc.parallel_loop(0, G // 16, unroll=8)
            def _sub(i):
                sl = pl.ds(i * 16, 16)
                dx = xs[sl] - xr[sl]
                dy = ys[sl] - yr[sl]
                dz = zs[sl] - zr[sl]
                sqv[sl] = dx * dx + dy * dy + dz * dz
            pltpu.sync_copy(sqv, sq_hbm.at[pl.ds(base, G)])
            return carry

        lax.fori_loop(0, NG, group, 0)

    k = pl.kernel(
        body,
        out_type=jax.ShapeDtypeStruct((EP,), jnp.float32),
        mesh=_sc_mesh(),
        compiler_params=pltpu.CompilerParams(use_tc_tiling_on_sc=False, needs_layout_passes=False),
        scratch_types=[
            pltpu.VMEM((G,), jnp.int32),
            pltpu.VMEM((G,), jnp.int32),
            pltpu.VMEM((G,), jnp.float32),
            pltpu.VMEM((G,), jnp.float32),
            pltpu.VMEM((G,), jnp.float32),
            pltpu.VMEM((G,), jnp.float32),
            pltpu.VMEM((G,), jnp.float32),
            pltpu.VMEM((G,), jnp.float32),
            pltpu.VMEM((G,), jnp.float32),
            pltpu.SemaphoreType.DMA,
        ],
    )
    return k(posx, posy, posz, senders, receivers)


def _msgpass_call(node_tab, senders, recv2d, rad, zeros_nf, width, edge_split,
                  G=G, dtype=jnp.float32):
    GC = G // 128
    ET = (EP // 2 if edge_split else EP) // NS
    SUPER = 1024
    NSUB = SUPER // G
    NG = ET // SUPER
    ROWS_T = N_PAD // NS
    WCH = max(w for w in (784, 448, 224, 112, 56)
              if w <= G and ROWS_T % w == 0)
    NWC = ROWS_T // WCH

    def body(node_hbm, snd_hbm, rcv_hbm, rad_hbm, zero_hbm, out_hbm,
             sidx, ridx2, rows, radv, shared, sem):
        c = lax.axis_index("c")
        s = lax.axis_index("s")

        for k2 in range(NWC):
            roff = s * ROWS_T + k2 * WCH
            pltpu.sync_copy(zero_hbm.at[pl.ds(roff, WCH)],
                            shared.at[pl.ds(roff, WCH)])
        plsc.subcore_barrier()

        if edge_split:
            ebase = c * (EP // 2) + s * ET
            rad_base0 = ebase
        else:
            ebase = s * ET
            rad_base0 = c * EP + s * ET

        def group(g, carry):
            base = ebase + g * SUPER
            rbase = rad_base0 + g * SUPER
            pltpu.sync_copy(snd_hbm.at[pl.ds(base, SUPER)], sidx)
            pltpu.sync_copy(
                rcv_hbm.at[pl.ds(pl.multiple_of(base // 128, 8), SUPER // 128)],
                ridx2)
            if not edge_split:
                off = c * N_PAD

                @plsc.parallel_loop(0, SUPER // 16, unroll=8)
                def _addoff(i):
                    sidx[pl.ds(i * 16, 16)] = sidx[pl.ds(i * 16, 16)] + off
            for sub in range(NSUB):
                gcp = pltpu.async_copy(
                    node_hbm.at[sidx.at[pl.ds(sub * G, G)]], rows, sem)
                pltpu.sync_copy(rad_hbm.at[pl.ds(rbase + sub * G, G)], radv)
                gcp.wait()

                if dtype == jnp.float32:
                    @plsc.parallel_loop(0, G, unroll=8)
                    def _mul(i):
                        for j in range(width // 16):
                            sl = pl.ds(j * 16, 16)
                            rows[i, sl] = rows[i, sl] * radv[i, sl]
                else:
                    @plsc.parallel_loop(0, G, unroll=8)
                    def _mul(i):
                        na, nb = plsc.unpack(rows[i, :],
                                             format=plsc.PackFormat.INTERLEAVED)
                        ra, rb = plsc.unpack(radv[i, :],
                                             format=plsc.PackFormat.INTERLEAVED)
                        rows[i, :] = plsc.pack(
                            na * ra, nb * rb,
                            format=plsc.PackFormat.INTERLEAVED)
                for j in range(GC):
                    pltpu.sync_copy(rows.at[pl.ds(j * 128, 128)],
                                    shared.at[ridx2.at[sub * GC + j]],
                                    add=True)
            return carry

        lax.fori_loop(0, NG, group, 0)
        plsc.subcore_barrier()
        for k2 in range(NWC):
            roff = s * ROWS_T + k2 * WCH
            pltpu.sync_copy(shared.at[pl.ds(roff, WCH)],
                            rows.at[pl.ds(0, WCH)])
            pltpu.sync_copy(rows.at[pl.ds(0, WCH)],
                            out_hbm.at[pl.ds(c * N_PAD + roff, WCH)])

    k = pl.kernel(
        body,
        out_type=jax.ShapeDtypeStruct((2 * N_PAD, width), dtype),
        mesh=_sc_mesh(),
        compiler_params=pltpu.CompilerParams(use_tc_tiling_on_sc=False, needs_layout_passes=False),
        scratch_types=[
            pltpu.VMEM((SUPER,), jnp.int32),
            pltpu.VMEM((SUPER // 128, 128), jnp.int32),
            pltpu.VMEM((G, width), dtype),
            pltpu.VMEM((G, width), dtype),
            pltpu.VMEM_SHARED((N_PAD, width), dtype),
            pltpu.SemaphoreType.DMA,
        ],
    )
    return k(node_tab, senders, recv2d, rad, zeros_nf)


def _radial_body(sq_ref, r0cat, r1diag, r2diag, rad0_ref, rad1_ref):
    for sb in range(8):
        x = sq_ref[sb:sb + 1, :]
        iszero = x == 0.0
        rn = jnp.where(iszero, 0.0, jnp.sqrt(jnp.where(iszero, 1.0, x)))
        xs = jnp.where(rn == 0.0, 1.0, rn)
        xb = jnp.broadcast_to(xs, (RB, BE))
        wm = (lax.broadcasted_iota(jnp.int32, (RB, BE), 0) + 1
              ).astype(jnp.float32) * np.pi
        bes = (2.0 / CUTOFF) * jnp.where(
            rn == 0.0, wm / CUTOFF, jnp.sin(wm * xb / CUTOFF) / xb)
        xh = rn * (1.0 / CUTOFF)
        x2 = xh * xh
        cut = (1.0 - 6.0 * x2 + 8.0 * x2 * xh - 3.0 * x2 * x2)
        cut = cut * jnp.where(xh < 1.0, 1.0, 0.0)
        rbT = bes * cut
        h1 = _silu(lax.dot_general(
            rbT, r0cat[...], (((0,), (0,)), ((), ())),
            preferred_element_type=jnp.float32))
        h2 = _silu(jnp.dot(h1.astype(jnp.bfloat16), r1diag[...],
                           preferred_element_type=jnp.float32))
        rad = jnp.dot(h2.astype(jnp.bfloat16), r2diag[...],
                      preferred_element_type=jnp.float32)
        sl = pl.ds(sb * BE, BE)
        rad0_ref[sl, :] = rad[:, :F0]
        rad1_ref[0, sl, :] = rad[:, F0:F0 + F1].astype(jnp.bfloat16)
        rad1_ref[1, sl, :] = rad[:, F0 + F1:].astype(jnp.bfloat16)


def _radial_call(sq2d, R0cat, R1diag, R2diag):
    grid = (EP // (8 * BE),)
    return pl.pallas_call(
        _radial_body,
        grid=grid,
        in_specs=[
            pl.BlockSpec((8, BE), lambda i: (i, 0)),
            pl.BlockSpec((RB, 2 * RM), lambda i: (0, 0)),
            pl.BlockSpec((2 * RM, 2 * RM), lambda i: (0, 0)),
            pl.BlockSpec((2 * RM, F0 + 2 * F1), lambda i: (0, 0)),
        ],
        out_specs=[
            pl.BlockSpec((8 * BE, F0), lambda i: (i, 0)),
            pl.BlockSpec((2, 8 * BE, F1), lambda i: (0, i, 0)),
        ],
        out_shape=[
            jax.ShapeDtypeStruct((EP, F0), jnp.float32),
            jax.ShapeDtypeStruct((2, EP, F1), jnp.bfloat16),
        ],
    )(sq2d, R0cat, R1diag, R2diag)


def _node0_body(spec_ref, w1_ref, out_ref):
    spec = spec_ref[...]
    oh = (spec == lax.broadcasted_iota(jnp.int32, (BN, N_SPECIES), 1))
    oh = oh.astype(jnp.float32)
    n0 = jnp.dot(oh, w1_ref[...], preferred_element_type=jnp.float32)
    out_ref[...] = jnp.pad(n0, ((0, 0), (0, F0 - N_SPECIES)))


def _node0_call(spec2d, W1_0):
    return pl.pallas_call(
        _node0_body,
        grid=(N_PAD // BN,),
        in_specs=[
            pl.BlockSpec((BN, 1), lambda i: (i, 0)),
            pl.BlockSpec((N_SPECIES, N_SPECIES), lambda i: (0, 0)),
        ],
        out_specs=pl.BlockSpec((BN, F0), lambda i: (i, 0)),
        out_shape=jax.ShapeDtypeStruct((N_PAD, F0), jnp.float32),
    )(spec2d, W1_0)


def _comb0_body(part_ref, spec_ref, skiptab_ref, w2_ref, w1n_ref,
                feat1_ref, node1_ref):
    p = part_ref[0] + part_ref[1]
    agg = p[:, :N_SPECIES] * (1.0 / np.sqrt(AVG_N))
    spec = spec_ref[...]
    oh = (spec == lax.broadcasted_iota(jnp.int32, (BN, N_SPECIES), 1))
    oh = oh.astype(jnp.float32)
    skip = jnp.dot(oh, skiptab_ref[...], preferred_element_type=jnp.float32)
    f1 = _silu(jnp.dot(agg, w2_ref[...],
                       preferred_element_type=jnp.float32) + skip)
    feat1_ref[...] = f1
    n1 = jnp.dot(f1, w1n_ref[...], preferred_element_type=jnp.float32)
    node1_ref[0] = n1[:, :F1].astype(jnp.bfloat16)
    node1_ref[1] = n1[:, F1:].astype(jnp.bfloat16)


def _comb0_call(part0, spec2d, skip0_tab, W2_0, W1_1):
    return pl.pallas_call(
        _comb0_body,
        grid=(N_PAD // BN,),
        in_specs=[
            pl.BlockSpec((2, BN, F0), lambda i: (0, i, 0)),
            pl.BlockSpec((BN, 1), lambda i: (i, 0)),
            pl.BlockSpec((N_SPECIES, HIDDEN), lambda i: (0, 0)),
            pl.BlockSpec((N_SPECIES, HIDDEN), lambda i: (0, 0)),
            pl.BlockSpec((HIDDEN, HIDDEN), lambda i: (0, 0)),
        ],
        out_specs=[
            pl.BlockSpec((BN, HIDDEN), lambda i: (i, 0)),
            pl.BlockSpec((2, BN, F1), lambda i: (0, i, 0)),
        ],
        out_shape=[
            jax.ShapeDtypeStruct((N_PAD, HIDDEN), jnp.float32),
            jax.ShapeDtypeStruct((2, N_PAD, F1), jnp.bfloat16),
        ],
    )(part0, spec2d, skip0_tab, W2_0, W1_1)


def _comb1_body(agg_ref, feat1_ref, spec_ref, wall_ref, w2_ref, wro_ref,
                ae_ref, out_ref):
    agg = jnp.concatenate([agg_ref[0], agg_ref[1]], axis=1)
    agg = agg.astype(jnp.float32) * (1.0 / np.sqrt(AVG_N))
    f1 = feat1_ref[...]
    hs = jnp.dot(f1, wall_ref[...], preferred_element_type=jnp.float32)
    spec = spec_ref[...]
    skip = jnp.zeros((BN, HIDDEN), jnp.float32)
    for s in range(N_SPECIES):
        skip = skip + jnp.where(spec == s,
                                hs[:, s * HIDDEN:(s + 1) * HIDDEN], 0.0)
    f2 = _silu(jnp.dot(agg, w2_ref[...],
                       preferred_element_type=jnp.float32) + skip)
    e = jnp.dot(f2, wro_ref[...], preferred_element_type=jnp.float32)
    e = e * SCALE + SHIFT
    oh = (spec == lax.broadcasted_iota(jnp.int32, (BN, N_SPECIES), 1))
    e = e + jnp.dot(oh.astype(jnp.float32), ae_ref[...],
                    preferred_element_type=jnp.float32)
    out_ref[...] = e


def _comb1_call(agg1, feat1, spec2d, W_all, W2_1, W_ro, ae2d):
    return pl.pallas_call(
        _comb1_body,
        grid=(N_PAD // BN,),
        in_specs=[
            pl.BlockSpec((2, BN, F1), lambda i: (0, i, 0)),
            pl.BlockSpec((BN, HIDDEN), lambda i: (i, 0)),
            pl.BlockSpec((BN, 1), lambda i: (i, 0)),
            pl.BlockSpec((HIDDEN, N_SPECIES * HIDDEN), lambda i: (0, 0)),
            pl.BlockSpec((HIDDEN, HIDDEN), lambda i: (0, 0)),
            pl.BlockSpec((HIDDEN, 1), lambda i: (0, 0)),
            pl.BlockSpec((N_SPECIES, 1), lambda i: (0, 0)),
        ],
        out_specs=pl.BlockSpec((BN, 1), lambda i: (i, 0)),
        out_shape=jax.ShapeDtypeStruct((N_PAD, 1), jnp.float32),
    )(agg1, feat1, spec2d, W_all, W2_1, W_ro, ae2d)


def kernel(positions, species, senders, receivers,
           l0_W1, l0_R0, l0_R1, l0_R2, l0_W2, l0_Wskip,
           l1_W1, l1_R0, l1_R1, l1_R2, l1_W2, l1_Wskip,
           W_ro, atom_energies):
    f32 = jnp.float32
    posx = jnp.zeros((N_PAD,), f32).at[:N].set(positions[:, 0].astype(f32))
    posy = jnp.zeros((N_PAD,), f32).at[:N].set(positions[:, 1].astype(f32))
    posz = jnp.zeros((N_PAD,), f32).at[:N].set(positions[:, 2].astype(f32))
    spec_p = jnp.zeros((N_PAD,), jnp.int32).at[:N].set(
        species.astype(jnp.int32))
    snd_p = jnp.full((EP,), TRASH, jnp.int32).at[:E].set(
        senders.astype(jnp.int32))
    rcv_p = jnp.full((EP,), TRASH, jnp.int32).at[:E].set(
        receivers.astype(jnp.int32))
    rcv2d = rcv_p.reshape(EP // 128, 128)
    spec2d = spec_p.reshape(N_PAD, 1)
    skip0_tab = l0_Wskip[jnp.arange(N_SPECIES), jnp.arange(N_SPECIES), :]
    W_all = jnp.transpose(l1_Wskip, (1, 0, 2)).reshape(
        HIDDEN, N_SPECIES * HIDDEN)
    ae2d = atom_energies.reshape(N_SPECIES, 1).astype(f32)
    zeros0 = jnp.zeros((N_PAD, F0), f32)
    zeros1 = jnp.zeros((N_PAD, F1), jnp.bfloat16)

    R0cat = jnp.concatenate([l0_R0, l1_R0], axis=1)
    R1diag = (jnp.zeros((2 * RM, 2 * RM), f32)
              .at[:RM, :RM].set(l0_R1)
              .at[RM:, RM:].set(l1_R1).astype(jnp.bfloat16))
    perm = np.empty((HIDDEN,), np.int32)
    perm[0:F1:2] = np.arange(0, 16)
    perm[1:F1:2] = np.arange(16, 32)
    perm[F1::2] = np.arange(32, 48)
    perm[F1 + 1::2] = np.arange(48, 64)
    R2diag = (jnp.zeros((2 * RM, F0 + 2 * F1), f32)
              .at[:RM, :RB].set(l0_R2)
              .at[RM:, F0:].set(l1_R2[:, perm]).astype(jnp.bfloat16))
    W1p = l1_W1[:, perm]
    W2p = l1_W2[perm, :]
    sq = _geom_call(posx, posy, posz, snd_p, rcv_p)
    rad0, rad1 = _radial_call(sq.reshape(EP // BE, BE), R0cat, R1diag, R2diag)
    node0 = _node0_call(spec2d, l0_W1)
    part0 = _msgpass_call(node0, snd_p, rcv2d, rad0, zeros0, F0,
                          edge_split=True).reshape(2, N_PAD, F0)
    feat1, node1 = _comb0_call(part0, spec2d, skip0_tab, l0_W2, W1p)
    agg1 = _msgpass_call(node1.reshape(2 * N_PAD, F1), snd_p, rcv2d,
                         rad1.reshape(2 * EP, F1), zeros1, F1,
                         edge_split=False, G=1024,
                         dtype=jnp.bfloat16).reshape(2, N_PAD, F1)
    e = _comb1_call(agg1, feat1, spec2d, W_all, W2p, W_ro, ae2d)
    return e[:N]


# speedup vs baseline: 4.3435x; 1.1029x over previous
import functools

import jax
import jax.numpy as jnp
import numpy as np
from jax import lax
from jax.experimental import pallas as pl
from jax.experimental.pallas import tpu as pltpu
from jax.experimental.pallas import tpu_sc as plsc

N_SPECIES = 8
HIDDEN = 64
RB = 8
RM = 64
CUTOFF = 5.0
AVG_N = 16.0
SCALE = 1.0
SHIFT = 0.0
N = 50000
E = 800000

N_PAD = 50176
EP = 819200
TRASH = N

NC = 2
NS = 16
G = 1024
GC = G // 128

F0 = 16
F1 = 32

BN = 1792
BE = 2048


def _silu(x):
    return x * (1.0 / (1.0 + jnp.exp(-x)))


def _sc_mesh():
    return plsc.VectorSubcoreMesh(core_axis_name="c", subcore_axis_name="s")


def _geom_call(pos4, senders, receivers):
    ET = EP // (NC * NS)
    NG = ET // G

    def body(pos_hbm, snd_hbm, rcv_hbm, sq_hbm,
             sidx, ridx, ps, pr, sqv, sem1, sem2):
        c = lax.axis_index("c")
        s = lax.axis_index("s")
        wid = s * NC + c
        base0 = wid * ET

        def group(g, carry):
            base = base0 + g * G
            pltpu.sync_copy(snd_hbm.at[pl.ds(base, G)], sidx)
            pltpu.sync_copy(rcv_hbm.at[pl.ds(base, G)], ridx)
            cp1 = pltpu.async_copy(pos_hbm.at[sidx], ps, sem1)
            cp2 = pltpu.async_copy(pos_hbm.at[ridx], pr, sem2)
            cp1.wait()
            cp2.wait()

            @plsc.parallel_loop(0, G // 16, unroll=8)
            def _sub(i):
                ids = i * 16 + lax.iota(jnp.int32, 16)
                acc = None
                for comp in range(3):
                    cv = jnp.full((16,), comp, jnp.int32)
                    a = plsc.load_gather(ps, [ids, cv])
                    b = plsc.load_gather(pr, [ids, cv])
                    d = a - b
                    acc = d * d if acc is None else acc + d * d
                sqv[pl.ds(i * 16, 16)] = acc

            pltpu.sync_copy(sqv, sq_hbm.at[pl.ds(base, G)])
            return carry

        lax.fori_loop(0, NG, group, 0)

    k = pl.kernel(
        body,
        out_type=jax.ShapeDtypeStruct((EP,), jnp.float32),
        mesh=_sc_mesh(),
        compiler_params=pltpu.CompilerParams(use_tc_tiling_on_sc=False, needs_layout_passes=False),
        scratch_types=[
            pltpu.VMEM((G,), jnp.int32),
            pltpu.VMEM((G,), jnp.int32),
            pltpu.VMEM((G, 4), jnp.float32),
            pltpu.VMEM((G, 4), jnp.float32),
            pltpu.VMEM((G,), jnp.float32),
            pltpu.SemaphoreType.DMA,
            pltpu.SemaphoreType.DMA,
        ],
    )
    return k(pos4, senders, receivers)


def _msgpass_call(node_tab, senders, recv2d, rad, zeros_nf, width, edge_split,
                  G=G, dtype=jnp.float32):
    GC = G // 128
    ET = (EP // 2 if edge_split else EP) // NS
    SUPER = 1024
    NSUB = SUPER // G
    NG = ET // SUPER
    ROWS_T = N_PAD // NS
    WCH = max(w for w in (784, 448, 224, 112, 56)
              if w <= G and ROWS_T % w == 0)
    NWC = ROWS_T // WCH

    def body(node_hbm, snd_hbm, rcv_hbm, rad_hbm, zero_hbm, out_hbm,
             sidx, ridx2, rows, radv, shared, sem):
        c = lax.axis_index("c")
        s = lax.axis_index("s")

        for k2 in range(NWC):
            roff = s * ROWS_T + k2 * WCH
            pltpu.sync_copy(zero_hbm.at[pl.ds(roff, WCH)],
                            shared.at[pl.ds(roff, WCH)])
        plsc.subcore_barrier()

        if edge_split:
            ebase = c * (EP // 2) + s * ET
            rad_base0 = ebase
        else:
            ebase = s * ET
            rad_base0 = c * EP + s * ET

        def group(g, carry):
            base = ebase + g * SUPER
            rbase = rad_base0 + g * SUPER
            pltpu.sync_copy(snd_hbm.at[pl.ds(base, SUPER)], sidx)
            pltpu.sync_copy(
                rcv_hbm.at[pl.ds(pl.multiple_of(base // 128, 8), SUPER // 128)],
                ridx2)
            if not edge_split:
                off = c * N_PAD

                @plsc.parallel_loop(0, SUPER // 16, unroll=8)
                def _addoff(i):
                    sidx[pl.ds(i * 16, 16)] = sidx[pl.ds(i * 16, 16)] + off
            for sub in range(NSUB):
                gcp = pltpu.async_copy(
                    node_hbm.at[sidx.at[pl.ds(sub * G, G)]], rows, sem)
                pltpu.sync_copy(rad_hbm.at[pl.ds(rbase + sub * G, G)], radv)
                gcp.wait()

                if dtype == jnp.float32:
                    @plsc.parallel_loop(0, G, unroll=8)
                    def _mul(i):
                        for j in range(width // 16):
                            sl = pl.ds(j * 16, 16)
                            rows[i, sl] = rows[i, sl] * radv[i, sl]
                else:
                    @plsc.parallel_loop(0, G, unroll=8)
                    def _mul(i):
                        na, nb = plsc.unpack(rows[i, :],
                                             format=plsc.PackFormat.INTERLEAVED)
                        ra, rb = plsc.unpack(radv[i, :],
                                             format=plsc.PackFormat.INTERLEAVED)
                        rows[i, :] = plsc.pack(
                            na * ra, nb * rb,
                            format=plsc.PackFormat.INTERLEAVED)
                for j in range(GC):
                    pltpu.sync_copy(rows.at[pl.ds(j * 128, 128)],
                                    shared.at[ridx2.at[sub * GC + j]],
                                    add=True)
            return carry

        lax.fori_loop(0, NG, group, 0)
        plsc.subcore_barrier()
        for k2 in range(NWC):
            roff = s * ROWS_T + k2 * WCH
            pltpu.sync_copy(shared.at[pl.ds(roff, WCH)],
                            rows.at[pl.ds(0, WCH)])
            pltpu.sync_copy(rows.at[pl.ds(0, WCH)],
                            out_hbm.at[pl.ds(c * N_PAD + roff, WCH)])

    k = pl.kernel(
        body,
        out_type=jax.ShapeDtypeStruct((2 * N_PAD, width), dtype),
        mesh=_sc_mesh(),
        compiler_params=pltpu.CompilerParams(use_tc_tiling_on_sc=False, needs_layout_passes=False),
        scratch_types=[
            pltpu.VMEM((SUPER,), jnp.int32),
            pltpu.VMEM((SUPER // 128, 128), jnp.int32),
            pltpu.VMEM((G, width), dtype),
            pltpu.VMEM((G, width), dtype),
            pltpu.VMEM_SHARED((N_PAD, width), dtype),
            pltpu.SemaphoreType.DMA,
        ],
    )
    return k(node_tab, senders, recv2d, rad, zeros_nf)


def _radial_body(sq_ref, r0cat, r1diag, r2diag, rad0_ref, rad1_ref):
    for sb in range(8):
        x = sq_ref[sb:sb + 1, :]
        iszero = x == 0.0
        rn = jnp.where(iszero, 0.0, jnp.sqrt(jnp.where(iszero, 1.0, x)))
        xs = jnp.where(rn == 0.0, 1.0, rn)
        xb = jnp.broadcast_to(xs, (RB, BE))
        wm = (lax.broadcasted_iota(jnp.int32, (RB, BE), 0) + 1
              ).astype(jnp.float32) * np.pi
        bes = (2.0 / CUTOFF) * jnp.where(
            rn == 0.0, wm / CUTOFF, jnp.sin(wm * xb / CUTOFF) / xb)
        xh = rn * (1.0 / CUTOFF)
        x2 = xh * xh
        cut = (1.0 - 6.0 * x2 + 8.0 * x2 * xh - 3.0 * x2 * x2)
        cut = cut * jnp.where(xh < 1.0, 1.0, 0.0)
        rbT = bes * cut
        h1 = _silu(lax.dot_general(
            rbT, r0cat[...], (((0,), (0,)), ((), ())),
            preferred_element_type=jnp.float32))
        h2 = _silu(jnp.dot(h1.astype(jnp.bfloat16), r1diag[...],
                           preferred_element_type=jnp.float32))
        rad = jnp.dot(h2.astype(jnp.bfloat16), r2diag[...],
                      preferred_element_type=jnp.float32)
        sl = pl.ds(sb * BE, BE)
        rad0_ref[sl, :] = rad[:, :F0]
        rad1_ref[0, sl, :] = rad[:, F0:F0 + F1].astype(jnp.bfloat16)
        rad1_ref[1, sl, :] = rad[:, F0 + F1:].astype(jnp.bfloat16)


def _radial_call(sq2d, R0cat, R1diag, R2diag):
    grid = (EP // (8 * BE),)
    return pl.pallas_call(
        _radial_body,
        grid=grid,
        in_specs=[
            pl.BlockSpec((8, BE), lambda i: (i, 0)),
            pl.BlockSpec((RB, 2 * RM), lambda i: (0, 0)),
            pl.BlockSpec((2 * RM, 2 * RM), lambda i: (0, 0)),
            pl.BlockSpec((2 * RM, F0 + 2 * F1), lambda i: (0, 0)),
        ],
        out_specs=[
            pl.BlockSpec((8 * BE, F0), lambda i: (i, 0)),
            pl.BlockSpec((2, 8 * BE, F1), lambda i: (0, i, 0)),
        ],
        out_shape=[
            jax.ShapeDtypeStruct((EP, F0), jnp.float32),
            jax.ShapeDtypeStruct((2, EP, F1), jnp.bfloat16),
        ],
    )(sq2d, R0cat, R1diag, R2diag)


def _node0_body(spec_ref, w1_ref, out_ref):
    spec = spec_ref[...]
    oh = (spec == lax.broadcasted_iota(jnp.int32, (BN, N_SPECIES), 1))
    oh = oh.astype(jnp.float32)
    n0 = jnp.dot(oh, w1_ref[...], preferred_element_type=jnp.float32)
    out_ref[...] = jnp.pad(n0, ((0, 0), (0, F0 - N_SPECIES)))


def _node0_call(spec2d, W1_0):
    return pl.pallas_call(
        _node0_body,
        grid=(N_PAD // BN,),
        in_specs=[
            pl.BlockSpec((BN, 1), lambda i: (i, 0)),
            pl.BlockSpec((N_SPECIES, N_SPECIES), lambda i: (0, 0)),
        ],
        out_specs=pl.BlockSpec((BN, F0), lambda i: (i, 0)),
        out_shape=jax.ShapeDtypeStruct((N_PAD, F0), jnp.float32),
    )(spec2d, W1_0)


def _comb0_body(part_ref, spec_ref, skiptab_ref, w2_ref, w1n_ref,
                feat1_ref, node1_ref):
    p = part_ref[0] + part_ref[1]
    agg = p[:, :N_SPECIES] * (1.0 / np.sqrt(AVG_N))
    spec = spec_ref[...]
    oh = (spec == lax.broadcasted_iota(jnp.int32, (BN, N_SPECIES), 1))
    oh = oh.astype(jnp.float32)
    skip = jnp.dot(oh, skiptab_ref[...], preferred_element_type=jnp.float32)
    f1 = _silu(jnp.dot(agg, w2_ref[...],
                       preferred_element_type=jnp.float32) + skip)
    feat1_ref[...] = f1
    n1 = jnp.dot(f1, w1n_ref[...], preferred_element_type=jnp.float32)
    node1_ref[0] = n1[:, :F1].astype(jnp.bfloat16)
    node1_ref[1] = n1[:, F1:].astype(jnp.bfloat16)


def _comb0_call(part0, spec2d, skip0_tab, W2_0, W1_1):
    return pl.pallas_call(
        _comb0_body,
        grid=(N_PAD // BN,),
        in_specs=[
            pl.BlockSpec((2, BN, F0), lambda i: (0, i, 0)),
            pl.BlockSpec((BN, 1), lambda i: (i, 0)),
            pl.BlockSpec((N_SPECIES, HIDDEN), lambda i: (0, 0)),
            pl.BlockSpec((N_SPECIES, HIDDEN), lambda i: (0, 0)),
            pl.BlockSpec((HIDDEN, HIDDEN), lambda i: (0, 0)),
        ],
        out_specs=[
            pl.BlockSpec((BN, HIDDEN), lambda i: (i, 0)),
            pl.BlockSpec((2, BN, F1), lambda i: (0, i, 0)),
        ],
        out_shape=[
            jax.ShapeDtypeStruct((N_PAD, HIDDEN), jnp.float32),
            jax.ShapeDtypeStruct((2, N_PAD, F1), jnp.bfloat16),
        ],
    )(part0, spec2d, skip0_tab, W2_0, W1_1)


def _comb1_body(agg_ref, feat1_ref, spec_ref, wall_ref, w2_ref, wro_ref,
                ae_ref, out_ref):
    agg = jnp.concatenate([agg_ref[0], agg_ref[1]], axis=1)
    agg = agg.astype(jnp.float32) * (1.0 / np.sqrt(AVG_N))
    f1 = feat1_ref[...]
    hs = jnp.dot(f1, wall_ref[...], preferred_element_type=jnp.float32)
    spec = spec_ref[...]
    skip = jnp.zeros((BN, HIDDEN), jnp.float32)
    for s in range(N_SPECIES):
        skip = skip + jnp.where(spec == s,
                                hs[:, s * HIDDEN:(s + 1) * HIDDEN], 0.0)
    f2 = _silu(jnp.dot(agg, w2_ref[...],
                       preferred_element_type=jnp.float32) + skip)
    e = jnp.dot(f2, wro_ref[...], preferred_element_type=jnp.float32)
    e = e * SCALE + SHIFT
    oh = (spec == lax.broadcasted_iota(jnp.int32, (BN, N_SPECIES), 1))
    e = e + jnp.dot(oh.astype(jnp.float32), ae_ref[...],
                    preferred_element_type=jnp.float32)
    out_ref[...] = e


def _comb1_call(agg1, feat1, spec2d, W_all, W2_1, W_ro, ae2d):
    return pl.pallas_call(
        _comb1_body,
        grid=(N_PAD // BN,),
        in_specs=[
            pl.BlockSpec((2, BN, F1), lambda i: (0, i, 0)),
            pl.BlockSpec((BN, HIDDEN), lambda i: (i, 0)),
            pl.BlockSpec((BN, 1), lambda i: (i, 0)),
            pl.BlockSpec((HIDDEN, N_SPECIES * HIDDEN), lambda i: (0, 0)),
            pl.BlockSpec((HIDDEN, HIDDEN), lambda i: (0, 0)),
            pl.BlockSpec((HIDDEN, 1), lambda i: (0, 0)),
            pl.BlockSpec((N_SPECIES, 1), lambda i: (0, 0)),
        ],
        out_specs=pl.BlockSpec((BN, 1), lambda i: (i, 0)),
        out_shape=jax.ShapeDtypeStruct((N_PAD, 1), jnp.float32),
    )(agg1, feat1, spec2d, W_all, W2_1, W_ro, ae2d)


def kernel(positions, species, senders, receivers,
           l0_W1, l0_R0, l0_R1, l0_R2, l0_W2, l0_Wskip,
           l1_W1, l1_R0, l1_R1, l1_R2, l1_W2, l1_Wskip,
           W_ro, atom_energies):
    f32 = jnp.float32
    pos4 = jnp.zeros((N_PAD, 4), f32).at[:N, :3].set(positions.astype(f32))
    spec_p = jnp.zeros((N_PAD,), jnp.int32).at[:N].set(
        species.astype(jnp.int32))
    snd_p = jnp.full((EP,), TRASH, jnp.int32).at[:E].set(
        senders.astype(jnp.int32))
    rcv_p = jnp.full((EP,), TRASH, jnp.int32).at[:E].set(
        receivers.astype(jnp.int32))
    rcv2d = rcv_p.reshape(EP // 128, 128)
    spec2d = spec_p.reshape(N_PAD, 1)
    skip0_tab = l0_Wskip[jnp.arange(N_SPECIES), jnp.arange(N_SPECIES), :]
    W_all = jnp.transpose(l1_Wskip, (1, 0, 2)).reshape(
        HIDDEN, N_SPECIES * HIDDEN)
    ae2d = atom_energies.reshape(N_SPECIES, 1).astype(f32)
    zeros0 = jnp.zeros((N_PAD, F0), f32)
    zeros1 = jnp.zeros((N_PAD, F1), jnp.bfloat16)

    R0cat = jnp.concatenate([l0_R0, l1_R0], axis=1)
    R1diag = (jnp.zeros((2 * RM, 2 * RM), f32)
              .at[:RM, :RM].set(l0_R1)
              .at[RM:, RM:].set(l1_R1).astype(jnp.bfloat16))
    perm = np.empty((HIDDEN,), np.int32)
    perm[0:F1:2] = np.arange(0, 16)
    perm[1:F1:2] = np.arange(16, 32)
    perm[F1::2] = np.arange(32, 48)
    perm[F1 + 1::2] = np.arange(48, 64)
    R2diag = (jnp.zeros((2 * RM, F0 + 2 * F1), f32)
              .at[:RM, :RB].set(l0_R2)
              .at[RM:, F0:].set(l1_R2[:, perm]).astype(jnp.bfloat16))
    W1p = l1_W1[:, perm]
    W2p = l1_W2[perm, :]
    sq = _geom_call(pos4, snd_p, rcv_p)
    rad0, rad1 = _radial_call(sq.reshape(EP // BE, BE), R0cat, R1diag, R2diag)
    node0 = _node0_call(spec2d, l0_W1)
    part0 = _msgpass_call(node0, snd_p, rcv2d, rad0, zeros0, F0,
                          edge_split=True).reshape(2, N_PAD, F0)
    feat1, node1 = _comb0_call(part0, spec2d, skip0_tab, l0_W2, W1p)
    agg1 = _msgpass_call(node1.reshape(2 * N_PAD, F1), snd_p, rcv2d,
                         rad1.reshape(2 * EP, F1), zeros1, F1,
                         edge_split=False, G=1024,
                         dtype=jnp.bfloat16).reshape(2, N_PAD, F1)
    e = _comb1_call(agg1, feat1, spec2d, W_all, W2p, W_ro, ae2d)
    return e[:N]
